# baseline (speedup 1.0000x reference)
import functools
import math

import jax
import jax.numpy as jnp
from jax import lax
from jax.experimental import pallas as pl
from jax.experimental.pallas import tpu as pltpu
from jax.experimental.pallas import tpu_sc as plsc

TOP_K = 8
N_WORKERS = 32
ROW_CHUNK = 256

_P = 0.3275911
_A1 = 0.254829592
_A2 = -0.284496736
_A3 = 1.421413741
_A4 = -1.453152027
_A5 = 1.061405429


def _mm_body(x1, x2, w1, w2, l_ref):
    acc = jax.lax.dot_general(
        x1[...], w1[...], (((1,), (1,)), ((), ())),
        preferred_element_type=jnp.float32,
    )
    l_ref[...] = acc + jax.lax.dot_general(
        x2[...], w2[...], (((1,), (1,)), ((), ())),
        preferred_element_type=jnp.float32,
    )


def _tc_logits_chunk(x, W, chunk_tokens, chunk_idx):
    hidden = x.shape[1]
    num_experts = W.shape[0]
    bt = 1024
    kc = hidden // 2
    blk0 = chunk_idx * (chunk_tokens // bt)
    return pl.pallas_call(
        _mm_body,
        grid=(chunk_tokens // bt,),
        in_specs=[
            pl.BlockSpec((bt, kc), lambda i: (blk0 + i, 0)),
            pl.BlockSpec((bt, kc), lambda i: (blk0 + i, 1)),
            pl.BlockSpec((num_experts, kc), lambda i: (0, 0)),
            pl.BlockSpec((num_experts, kc), lambda i: (0, 1)),
        ],
        out_specs=pl.BlockSpec((bt, num_experts), lambda i: (i, 0)),
        out_shape=jax.ShapeDtypeStruct((chunk_tokens, num_experts), jnp.float32),
    )(x, x, W, W)


def _sort_asc(v):
    return jnp.sort(v)


def _rev(v):
    return lax.rev(v, (0,))


def _top16_merge(a_sorted, b_sorted):
    return jnp.maximum(_rev(a_sorted), b_sorted)


def _erf(u):
    a = jnp.abs(u)
    t = 1.0 / (1.0 + _P * a)
    y = t * (_A1 + t * (_A2 + t * (_A3 + t * (_A4 + t * _A5))))
    r = y * jnp.exp(-(a * a))
    return jnp.where(u < 0.0, r - 1.0, 1.0 - r)


def _sc_router(n_tokens, num_experts, noise_row0=0):
    sigma = 1.0 / num_experts
    rows_pw = n_tokens // N_WORKERS
    n_chunks = rows_pw // ROW_CHUNK
    mesh = plsc.VectorSubcoreMesh(core_axis_name="c", subcore_axis_name="s")

    out_type = (
        jax.ShapeDtypeStruct((n_tokens * TOP_K,), jnp.float32),
        jax.ShapeDtypeStruct((n_tokens * TOP_K,), jnp.int32),
        jax.ShapeDtypeStruct((n_tokens * num_experts,), jnp.float32),
        jax.ShapeDtypeStruct((n_tokens * num_experts,), jnp.float32),
    )
    scratch_types = [
        pltpu.VMEM((ROW_CHUNK * num_experts,), jnp.float32),
        pltpu.VMEM((ROW_CHUNK * num_experts,), jnp.float32),
        pltpu.VMEM((ROW_CHUNK * num_experts,), jnp.float32),
        pltpu.VMEM((ROW_CHUNK * num_experts,), jnp.float32),
        pltpu.VMEM((ROW_CHUNK * TOP_K,), jnp.float32),
        pltpu.VMEM((ROW_CHUNK * TOP_K,), jnp.int32),
    ]

    @functools.partial(
        pl.kernel, mesh=mesh, out_type=out_type, scratch_types=scratch_types,
        compiler_params=pltpu.CompilerParams(needs_layout_passes=False),
    )
    def sc_kernel(l_hbm, n_hbm, w_hbm, i_hbm, g_hbm, lp_hbm,
                  lv, nv, gv, lpv, wv, iv):
        wid = lax.axis_index("s") * 2 + lax.axis_index("c")
        iota16 = lax.iota(jnp.int32, 16)
        mask8 = iota16 < 8
        zero16 = jnp.zeros((16,), jnp.float32)
        lane8 = jnp.full((16,), 8, jnp.int32)

        dnums = lax.GatherDimensionNumbers(
            offset_dims=(), collapsed_slice_dims=(0,), start_index_map=(0,)
        )

        def _perm(v, idx):
            return lax.gather(
                v, idx[:, None], dnums, (1,),
                mode=lax.GatherScatterMode.PROMISE_IN_BOUNDS,
            )

        def _bfly_max(v):
            for k in (1, 2, 4, 8):
                v = jnp.maximum(v, _perm(v, iota16 ^ k))
            return v

        def _bfly_sum(v):
            for k in (1, 2, 4, 8):
                v = v + _perm(v, iota16 ^ k)
            return v

        def do_chunk(c, _):
            base = (wid * rows_pw + c * ROW_CHUNK) * num_experts
            obase = (wid * rows_pw + c * ROW_CHUNK) * TOP_K
            pltpu.sync_copy(l_hbm.at[pl.ds(base, ROW_CHUNK * num_experts)], lv)
            pltpu.sync_copy(
                n_hbm.at[
                    pl.ds(noise_row0 * num_experts + base,
                          ROW_CHUNK * num_experts)
                ],
                nv,
            )

            def one_row(r):
                bi = r * num_experts
                ls = [lv[pl.ds(bi + 16 * j, 16)] for j in range(4)]
                mm = _bfly_max(
                    jnp.maximum(
                        jnp.maximum(ls[0], ls[1]), jnp.maximum(ls[2], ls[3])
                    )
                )
                es = [jnp.exp(l - mm) for l in ls]
                s = _bfly_sum(es[0] + es[1] + es[2] + es[3])
                inv_s = 1.0 / s
                for j in range(4):
                    gv[pl.ds(bi + 16 * j, 16)] = es[j] * inv_s

                ss = []
                for j in range(4):
                    b = lax.bitcast_convert_type(ls[j], jnp.int32)
                    gidx = iota16 + (16 * j)
                    low = jnp.where(b < 0, gidx, 63 - gidx)
                    key = lax.bitcast_convert_type(
                        (b & jnp.int32(~63)) | low, jnp.float32
                    )
                    ss.append(_sort_asc(key))
                u01 = _sort_asc(_top16_merge(ss[0], ss[1]))
                u23 = _sort_asc(_top16_merge(ss[2], ss[3]))
                top = _rev(_sort_asc(_top16_merge(u01, u23)))
                tb = lax.bitcast_convert_type(top, jnp.int32)
                lowb = tb & jnp.int32(63)
                idx = jnp.where(tb < 0, lowb, jnp.int32(63) - lowb)
                v8 = lax.bitcast_convert_type(tb & jnp.int32(~63), jnp.float32)
                e8 = jnp.where(mask8, jnp.exp(v8 - mm), zero16)
                norm = 1.0 / (_bfly_sum(e8) + s * 1e-09)
                w8 = e8 * norm
                i8 = jnp.where(mask8, idx, 0)

                qs = [
                    _sort_asc(ls[j] + nv[pl.ds(bi + 16 * j, 16)] * sigma)
                    for j in range(4)
                ]
                r01 = _sort_asc(_top16_merge(qs[0], qs[1]))
                r23 = _sort_asc(_top16_merge(qs[2], qs[3]))
                zf = _sort_asc(_top16_merge(r01, r23))
                tau = _perm(zf, lane8)
                for j in range(4):
                    u = (tau - ls[j]) * (num_experts / math.sqrt(2.0))
                    lpv[pl.ds(bi + 16 * j, 16)] = 0.5 * (1.0 - _erf(u))
                return w8, i8

            lo8 = iota16 & 7

            def do_pair(p, _):
                w_a, i_a = one_row(2 * p)
                w_b, i_b = one_row(2 * p + 1)
                wv[pl.ds(p * 16, 16)] = jnp.where(mask8, w_a, _perm(w_b, lo8))
                iv[pl.ds(p * 16, 16)] = jnp.where(mask8, i_a, _perm(i_b, lo8))
                return 0

            lax.fori_loop(0, ROW_CHUNK // 2, do_pair, 0)
            pltpu.sync_copy(gv, g_hbm.at[pl.ds(base, ROW_CHUNK * num_experts)])
            pltpu.sync_copy(
                lpv, lp_hbm.at[pl.ds(base, ROW_CHUNK * num_experts)]
            )
            pltpu.sync_copy(wv, w_hbm.at[pl.ds(obase, ROW_CHUNK * TOP_K)])
            pltpu.sync_copy(iv, i_hbm.at[pl.ds(obase, ROW_CHUNK * TOP_K)])
            return 0

        lax.fori_loop(0, n_chunks, do_chunk, 0)

    return sc_kernel


def kernel(x, W, noise):
    n_tokens, hidden = x.shape
    num_experts = W.shape[0]
    n_chunks = 4
    ch = n_tokens // n_chunks
    noise_flat = noise.reshape(-1)
    parts = []
    for c in range(n_chunks):
        lg = _tc_logits_chunk(x, W, ch, c)
        sc = _sc_router(ch, num_experts, noise_row0=c * ch)
        wp, ip, gf, lpf = sc(lg.reshape(-1), noise_flat)
        parts.append((lg, wp, ip, gf, lpf))
    logits = jnp.concatenate([p[0] for p in parts], axis=0)
    tkw = jnp.concatenate(
        [p[1].reshape(ch, TOP_K) for p in parts], axis=0
    )
    tki = jnp.concatenate(
        [p[2].reshape(ch, TOP_K) for p in parts], axis=0
    )
    g = jnp.concatenate(
        [p[3].reshape(ch, num_experts) for p in parts], axis=0
    )
    lp = jnp.concatenate(
        [p[4].reshape(ch, num_experts) for p in parts], axis=0
    )
    return (tkw, tki, g, logits, lp, tki)


# speedup vs baseline: 1.0016x; 1.0016x over previous
import functools
import math

import jax
import jax.numpy as jnp
from jax import lax
from jax.experimental import pallas as pl
from jax.experimental.pallas import tpu as pltpu
from jax.experimental.pallas import tpu_sc as plsc

TOP_K = 8
N_WORKERS = 32
ROW_CHUNK = 256

_P = 0.3275911
_A1 = 0.254829592
_A2 = -0.284496736
_A3 = 1.421413741
_A4 = -1.453152027
_A5 = 1.061405429


def _mm_body(x1, x2, w1, w2, l_ref, g_ref):
    acc = jax.lax.dot_general(
        x1[...], w1[...], (((1,), (1,)), ((), ())),
        preferred_element_type=jnp.float32,
    )
    logits = acc + jax.lax.dot_general(
        x2[...], w2[...], (((1,), (1,)), ((), ())),
        preferred_element_type=jnp.float32,
    )
    l_ref[...] = logits
    m = jnp.max(logits, axis=1, keepdims=True)
    e = jnp.exp(logits - m)
    g_ref[...] = e / jnp.sum(e, axis=1, keepdims=True)


def _tc_logits(x, W):
    n_tokens, hidden = x.shape
    num_experts = W.shape[0]
    bt = 1024
    kc = hidden // 2
    return pl.pallas_call(
        _mm_body,
        grid=(n_tokens // bt,),
        in_specs=[
            pl.BlockSpec((bt, kc), lambda i: (i, 0)),
            pl.BlockSpec((bt, kc), lambda i: (i, 1)),
            pl.BlockSpec((num_experts, kc), lambda i: (0, 0)),
            pl.BlockSpec((num_experts, kc), lambda i: (0, 1)),
        ],
        out_specs=(
            pl.BlockSpec((bt, num_experts), lambda i: (i, 0)),
            pl.BlockSpec((bt, num_experts), lambda i: (i, 0)),
        ),
        out_shape=(
            jax.ShapeDtypeStruct((n_tokens, num_experts), jnp.float32),
            jax.ShapeDtypeStruct((n_tokens, num_experts), jnp.float32),
        ),
    )(x, x, W, W)


def _sort_asc(v):
    return jnp.sort(v)


def _rev(v):
    return lax.rev(v, (0,))


def _top16_merge(a_sorted, b_sorted):
    return jnp.maximum(_rev(a_sorted), b_sorted)


def _erf(u):
    a = jnp.abs(u)
    t = 1.0 / (1.0 + _P * a)
    y = t * (_A1 + t * (_A2 + t * (_A3 + t * (_A4 + t * _A5))))
    r = y * jnp.exp(-(a * a))
    return jnp.where(u < 0.0, r - 1.0, 1.0 - r)


def _sc_router(n_tokens, num_experts, noise_row0=0):
    sigma = 1.0 / num_experts
    rows_pw = n_tokens // N_WORKERS
    n_chunks = rows_pw // ROW_CHUNK
    mesh = plsc.VectorSubcoreMesh(core_axis_name="c", subcore_axis_name="s")

    out_type = (
        jax.ShapeDtypeStruct((n_tokens * TOP_K,), jnp.float32),
        jax.ShapeDtypeStruct((n_tokens * TOP_K,), jnp.int32),
        jax.ShapeDtypeStruct((n_tokens * num_experts,), jnp.float32),
    )
    scratch_types = [
        pltpu.VMEM((ROW_CHUNK * num_experts,), jnp.float32),
        pltpu.VMEM((ROW_CHUNK * num_experts,), jnp.float32),
        pltpu.VMEM((ROW_CHUNK * num_experts,), jnp.float32),
        pltpu.VMEM((ROW_CHUNK * TOP_K,), jnp.float32),
        pltpu.VMEM((ROW_CHUNK * TOP_K,), jnp.int32),
    ]

    @functools.partial(
        pl.kernel, mesh=mesh, out_type=out_type, scratch_types=scratch_types,
        compiler_params=pltpu.CompilerParams(needs_layout_passes=False),
    )
    def sc_kernel(l_hbm, n_hbm, w_hbm, i_hbm, lp_hbm,
                  lv, nv, lpv, wv, iv):
        wid = lax.axis_index("s") * 2 + lax.axis_index("c")
        iota16 = lax.iota(jnp.int32, 16)
        mask8 = iota16 < 8
        zero16 = jnp.zeros((16,), jnp.float32)
        lane8 = jnp.full((16,), 8, jnp.int32)

        dnums = lax.GatherDimensionNumbers(
            offset_dims=(), collapsed_slice_dims=(0,), start_index_map=(0,)
        )

        def _perm(v, idx):
            return lax.gather(
                v, idx[:, None], dnums, (1,),
                mode=lax.GatherScatterMode.PROMISE_IN_BOUNDS,
            )

        def _bfly_max(v):
            for k in (1, 2, 4, 8):
                v = jnp.maximum(v, _perm(v, iota16 ^ k))
            return v

        def _bfly_sum(v):
            for k in (1, 2, 4, 8):
                v = v + _perm(v, iota16 ^ k)
            return v

        def do_chunk(c, _):
            base = (wid * rows_pw + c * ROW_CHUNK) * num_experts
            obase = (wid * rows_pw + c * ROW_CHUNK) * TOP_K
            pltpu.sync_copy(l_hbm.at[pl.ds(base, ROW_CHUNK * num_experts)], lv)
            pltpu.sync_copy(
                n_hbm.at[
                    pl.ds(noise_row0 * num_experts + base,
                          ROW_CHUNK * num_experts)
                ],
                nv,
            )

            def one_row(r):
                bi = r * num_experts
                ls = [lv[pl.ds(bi + 16 * j, 16)] for j in range(4)]
                mm = _bfly_max(
                    jnp.maximum(
                        jnp.maximum(ls[0], ls[1]), jnp.maximum(ls[2], ls[3])
                    )
                )
                es = [jnp.exp(l - mm) for l in ls]
                s = _bfly_sum(es[0] + es[1] + es[2] + es[3])

                ss = []
                for j in range(4):
                    b = lax.bitcast_convert_type(ls[j], jnp.int32)
                    gidx = iota16 + (16 * j)
                    low = jnp.where(b < 0, gidx, 63 - gidx)
                    key = lax.bitcast_convert_type(
                        (b & jnp.int32(~63)) | low, jnp.float32
                    )
                    ss.append(_sort_asc(key))
                u01 = _sort_asc(_top16_merge(ss[0], ss[1]))
                u23 = _sort_asc(_top16_merge(ss[2], ss[3]))
                top = _rev(_sort_asc(_top16_merge(u01, u23)))
                tb = lax.bitcast_convert_type(top, jnp.int32)
                lowb = tb & jnp.int32(63)
                idx = jnp.where(tb < 0, lowb, jnp.int32(63) - lowb)
                v8 = lax.bitcast_convert_type(tb & jnp.int32(~63), jnp.float32)
                e8 = jnp.where(mask8, jnp.exp(v8 - mm), zero16)
                norm = 1.0 / (_bfly_sum(e8) + s * 1e-09)
                w8 = e8 * norm
                i8 = jnp.where(mask8, idx, 0)

                qs = [
                    _sort_asc(ls[j] + nv[pl.ds(bi + 16 * j, 16)] * sigma)
                    for j in range(4)
                ]
                r01 = _sort_asc(_top16_merge(qs[0], qs[1]))
                r23 = _sort_asc(_top16_merge(qs[2], qs[3]))
                zf = _sort_asc(_top16_merge(r01, r23))
                tau = _perm(zf, lane8)
                for j in range(4):
                    u = (tau - ls[j]) * (num_experts / math.sqrt(2.0))
                    lpv[pl.ds(bi + 16 * j, 16)] = 0.5 * (1.0 - _erf(u))
                return w8, i8

            lo8 = iota16 & 7

            def do_pair(p, _):
                w_a, i_a = one_row(2 * p)
                w_b, i_b = one_row(2 * p + 1)
                wv[pl.ds(p * 16, 16)] = jnp.where(mask8, w_a, _perm(w_b, lo8))
                iv[pl.ds(p * 16, 16)] = jnp.where(mask8, i_a, _perm(i_b, lo8))
                return 0

            lax.fori_loop(0, ROW_CHUNK // 2, do_pair, 0)
            pltpu.sync_copy(
                lpv, lp_hbm.at[pl.ds(base, ROW_CHUNK * num_experts)]
            )
            pltpu.sync_copy(wv, w_hbm.at[pl.ds(obase, ROW_CHUNK * TOP_K)])
            pltpu.sync_copy(iv, i_hbm.at[pl.ds(obase, ROW_CHUNK * TOP_K)])
            return 0

        lax.fori_loop(0, n_chunks, do_chunk, 0)

    return sc_kernel


def kernel(x, W, noise):
    n_tokens, hidden = x.shape
    num_experts = W.shape[0]
    logits, g = _tc_logits(x, W)
    sc = _sc_router(n_tokens, num_experts)
    wp, ip, lpf = sc(logits.reshape(-1), noise.reshape(-1))
    tkw = wp.reshape(n_tokens, TOP_K)
    tki = ip.reshape(n_tokens, TOP_K)
    lp = lpf.reshape(n_tokens, num_experts)
    return (tkw, tki, g, logits, lp, tki)


# speedup vs baseline: 1.0354x; 1.0338x over previous
import functools
import math

import jax
import jax.numpy as jnp
from jax import lax
from jax.experimental import pallas as pl
from jax.experimental.pallas import tpu as pltpu
from jax.experimental.pallas import tpu_sc as plsc

TOP_K = 8
N_WORKERS = 32
ROW_CHUNK = 256

_P = 0.3275911
_A1 = 0.254829592
_A2 = -0.284496736
_A3 = 1.421413741
_A4 = -1.453152027
_A5 = 1.061405429


def _mm_body(x1, x2, w1, w2, l_ref):
    acc = jax.lax.dot_general(
        x1[...], w1[...], (((1,), (1,)), ((), ())),
        preferred_element_type=jnp.float32,
    )
    l_ref[...] = acc + jax.lax.dot_general(
        x2[...], w2[...], (((1,), (1,)), ((), ())),
        preferred_element_type=jnp.float32,
    )


def _tc_logits_chunk(x, W, chunk_tokens, chunk_idx):
    hidden = x.shape[1]
    num_experts = W.shape[0]
    bt = 1024
    kc = hidden // 2
    blk0 = chunk_idx * (chunk_tokens // bt)
    return pl.pallas_call(
        _mm_body,
        grid=(chunk_tokens // bt,),
        in_specs=[
            pl.BlockSpec((bt, kc), lambda i: (blk0 + i, 0)),
            pl.BlockSpec((bt, kc), lambda i: (blk0 + i, 1)),
            pl.BlockSpec((num_experts, kc), lambda i: (0, 0)),
            pl.BlockSpec((num_experts, kc), lambda i: (0, 1)),
        ],
        out_specs=pl.BlockSpec((bt, num_experts), lambda i: (i, 0)),
        out_shape=jax.ShapeDtypeStruct((chunk_tokens, num_experts), jnp.float32),
    )(x, x, W, W)


def _sort_asc(v):
    return jnp.sort(v)


def _rev(v):
    return lax.rev(v, (0,))


def _top16_merge(a_sorted, b_sorted):
    return jnp.maximum(_rev(a_sorted), b_sorted)


def _erf(u):
    a = jnp.abs(u)
    t = 1.0 / (1.0 + _P * a)
    y = t * (_A1 + t * (_A2 + t * (_A3 + t * (_A4 + t * _A5))))
    r = y * jnp.exp(-(a * a))
    return jnp.where(u < 0.0, r - 1.0, 1.0 - r)


def _sc_router(n_tokens, num_experts, noise_row0=0):
    sigma = 1.0 / num_experts
    rows_pw = n_tokens // N_WORKERS
    n_chunks = rows_pw // ROW_CHUNK
    mesh = plsc.VectorSubcoreMesh(core_axis_name="c", subcore_axis_name="s")

    out_type = (
        jax.ShapeDtypeStruct((n_tokens * TOP_K,), jnp.float32),
        jax.ShapeDtypeStruct((n_tokens * TOP_K,), jnp.int32),
        jax.ShapeDtypeStruct((n_tokens * num_experts,), jnp.float32),
        jax.ShapeDtypeStruct((n_tokens * num_experts,), jnp.float32),
    )
    scratch_types = [
        pltpu.VMEM((ROW_CHUNK * num_experts,), jnp.float32),
        pltpu.VMEM((ROW_CHUNK * num_experts,), jnp.float32),
        pltpu.VMEM((ROW_CHUNK * num_experts,), jnp.float32),
        pltpu.VMEM((ROW_CHUNK * num_experts,), jnp.float32),
        pltpu.VMEM((ROW_CHUNK * TOP_K,), jnp.float32),
        pltpu.VMEM((ROW_CHUNK * TOP_K,), jnp.int32),
    ]

    @functools.partial(
        pl.kernel, mesh=mesh, out_type=out_type, scratch_types=scratch_types,
        compiler_params=pltpu.CompilerParams(needs_layout_passes=False),
    )
    def sc_kernel(l_hbm, n_hbm, w_hbm, i_hbm, g_hbm, lp_hbm,
                  lv, nv, gv, lpv, wv, iv):
        wid = lax.axis_index("s") * 2 + lax.axis_index("c")
        iota16 = lax.iota(jnp.int32, 16)
        mask8 = iota16 < 8
        zero16 = jnp.zeros((16,), jnp.float32)
        lane8 = jnp.full((16,), 8, jnp.int32)

        dnums = lax.GatherDimensionNumbers(
            offset_dims=(), collapsed_slice_dims=(0,), start_index_map=(0,)
        )

        def _perm(v, idx):
            return lax.gather(
                v, idx[:, None], dnums, (1,),
                mode=lax.GatherScatterMode.PROMISE_IN_BOUNDS,
            )

        def _bfly_max(v):
            for k in (1, 2, 4, 8):
                v = jnp.maximum(v, _perm(v, iota16 ^ k))
            return v

        def _bfly_sum(v):
            for k in (1, 2, 4, 8):
                v = v + _perm(v, iota16 ^ k)
            return v

        def do_chunk(c, _):
            base = (wid * rows_pw + c * ROW_CHUNK) * num_experts
            obase = (wid * rows_pw + c * ROW_CHUNK) * TOP_K
            pltpu.sync_copy(l_hbm.at[pl.ds(base, ROW_CHUNK * num_experts)], lv)
            pltpu.sync_copy(
                n_hbm.at[
                    pl.ds(noise_row0 * num_experts + base,
                          ROW_CHUNK * num_experts)
                ],
                nv,
            )

            def one_row(r):
                bi = r * num_experts
                ls = [lv[pl.ds(bi + 16 * j, 16)] for j in range(4)]
                mm = _bfly_max(
                    jnp.maximum(
                        jnp.maximum(ls[0], ls[1]), jnp.maximum(ls[2], ls[3])
                    )
                )
                es = [jnp.exp(l - mm) for l in ls]
                s = _bfly_sum(es[0] + es[1] + es[2] + es[3])
                inv_s = 1.0 / s
                for j in range(4):
                    gv[pl.ds(bi + 16 * j, 16)] = es[j] * inv_s

                ss = []
                for j in range(4):
                    b = lax.bitcast_convert_type(ls[j], jnp.int32)
                    gidx = iota16 + (16 * j)
                    low = jnp.where(b < 0, gidx, 63 - gidx)
                    key = lax.bitcast_convert_type(
                        (b & jnp.int32(~63)) | low, jnp.float32
                    )
                    ss.append(_sort_asc(key))
                u01 = _sort_asc(_top16_merge(ss[0], ss[1]))
                u23 = _sort_asc(_top16_merge(ss[2], ss[3]))
                top = _rev(_sort_asc(_top16_merge(u01, u23)))
                tb = lax.bitcast_convert_type(top, jnp.int32)
                lowb = tb & jnp.int32(63)
                idx = jnp.where(tb < 0, lowb, jnp.int32(63) - lowb)
                v8 = lax.bitcast_convert_type(tb & jnp.int32(~63), jnp.float32)
                e8 = jnp.where(mask8, jnp.exp(v8 - mm), zero16)
                norm = 1.0 / (_bfly_sum(e8) + s * 1e-09)
                w8 = e8 * norm
                i8 = jnp.where(mask8, idx, 0)

                qs = [
                    _sort_asc(ls[j] + nv[pl.ds(bi + 16 * j, 16)] * sigma)
                    for j in range(4)
                ]
                r01 = _sort_asc(_top16_merge(qs[0], qs[1]))
                r23 = _sort_asc(_top16_merge(qs[2], qs[3]))
                zf = _sort_asc(_top16_merge(r01, r23))
                tau = _perm(zf, lane8)
                for j in range(4):
                    u = (tau - ls[j]) * (num_experts / math.sqrt(2.0))
                    lpv[pl.ds(bi + 16 * j, 16)] = 0.5 * (1.0 - _erf(u))
                return w8, i8

            lo8 = iota16 & 7

            def do_pair(p, _):
                w_a, i_a = one_row(2 * p)
                w_b, i_b = one_row(2 * p + 1)
                wv[pl.ds(p * 16, 16)] = jnp.where(mask8, w_a, _perm(w_b, lo8))
                iv[pl.ds(p * 16, 16)] = jnp.where(mask8, i_a, _perm(i_b, lo8))
                return 0

            lax.fori_loop(0, ROW_CHUNK // 2, do_pair, 0)
            pltpu.sync_copy(gv, g_hbm.at[pl.ds(base, ROW_CHUNK * num_experts)])
            pltpu.sync_copy(
                lpv, lp_hbm.at[pl.ds(base, ROW_CHUNK * num_experts)]
            )
            pltpu.sync_copy(wv, w_hbm.at[pl.ds(obase, ROW_CHUNK * TOP_K)])
            pltpu.sync_copy(iv, i_hbm.at[pl.ds(obase, ROW_CHUNK * TOP_K)])
            return 0

        lax.fori_loop(0, n_chunks, do_chunk, 0)

    return sc_kernel


def kernel(x, W, noise):
    n_tokens, hidden = x.shape
    num_experts = W.shape[0]
    n_chunks = 1
    ch = n_tokens // n_chunks
    noise_flat = noise.reshape(-1)
    parts = []
    for c in range(n_chunks):
        lg = _tc_logits_chunk(x, W, ch, c)
        sc = _sc_router(ch, num_experts, noise_row0=c * ch)
        wp, ip, gf, lpf = sc(lg.reshape(-1), noise_flat)
        parts.append((lg, wp, ip, gf, lpf))
    logits = jnp.concatenate([p[0] for p in parts], axis=0)
    tkw = jnp.concatenate(
        [p[1].reshape(ch, TOP_K) for p in parts], axis=0
    )
    tki = jnp.concatenate(
        [p[2].reshape(ch, TOP_K) for p in parts], axis=0
    )
    g = jnp.concatenate(
        [p[3].reshape(ch, num_experts) for p in parts], axis=0
    )
    lp = jnp.concatenate(
        [p[4].reshape(ch, num_experts) for p in parts], axis=0
    )
    return (tkw, tki, g, logits, lp, tki)
